# baseline (speedup 1.0000x reference)
import functools

import jax
import jax.numpy as jnp
from jax import lax
from jax.experimental import pallas as pl
from jax.experimental.pallas import tpu as pltpu
from jax.experimental.pallas import tpu_sc as plsc

EMBED = 32
NC, NS = 2, 16
NW = NC * NS


@functools.lru_cache(maxsize=None)
def _make_gather(batch: int, hist: int, g_rows: int):
    rows_per_w = batch // NW
    steps = rows_per_w // g_rows
    assert steps * g_rows == rows_per_w and rows_per_w * NW == batch

    mesh = plsc.VectorSubcoreMesh(
        core_axis_name="c", subcore_axis_name="s",
        num_cores=NC, num_subcores=NS)

    @functools.partial(
        pl.kernel,
        out_type=jax.ShapeDtypeStruct((batch, hist, EMBED), jnp.float32),
        mesh=mesh,
        scratch_types=[
            pltpu.VMEM((2, g_rows, hist), jnp.int32),
            pltpu.VMEM((2, g_rows, hist, EMBED), jnp.float32),
            pltpu.SemaphoreType.DMA,
            pltpu.SemaphoreType.DMA,
            pltpu.SemaphoreType.DMA,
            pltpu.SemaphoreType.DMA,
        ],
        compiler_params=pltpu.CompilerParams(use_tc_tiling_on_sc=False),
    )
    def grab(idx_hbm, tab_hbm, out_hbm, idx_v, rows_v, g0, g1, o0, o1):
        wid = lax.axis_index("s") * NC + lax.axis_index("c")
        base = wid * rows_per_w
        gsem = (g0, g1)
        osem = (o0, o1)

        def load_idx(g, b):
            pltpu.sync_copy(idx_hbm.at[pl.ds(base + g * g_rows, g_rows)],
                            idx_v.at[b])

        def fire_gather(b):
            return [
                pltpu.async_copy(tab_hbm.at[idx_v.at[b, r]],
                                 rows_v.at[b, r], gsem[b])
                for r in range(g_rows)
            ]

        def fire_out(g, b):
            return pltpu.async_copy(
                rows_v.at[b],
                out_hbm.at[pl.ds(base + g * g_rows, g_rows)],
                osem[b])

        load_idx(0, 0)
        gds = {0: fire_gather(0)}
        ods = {}
        for g in range(steps):
            b = g % 2
            nb = (g + 1) % 2
            if g + 1 < steps:
                load_idx(g + 1, nb)
                if g >= 1:
                    ods.pop(nb).wait()
                gds[nb] = fire_gather(nb)
            for d in gds.pop(b):
                d.wait()
            ods[b] = fire_out(g, b)
        for d in ods.values():
            d.wait()

    return grab


def kernel(inputs, embeddings):
    batch, hist = inputs.shape
    idx = inputs if inputs.dtype == jnp.int32 else inputs.astype(jnp.int32)
    return _make_gather(batch, hist, 16)(idx, embeddings)


# speedup vs baseline: 1.0535x; 1.0535x over previous
import functools

import jax
import jax.numpy as jnp
from jax import lax
from jax.experimental import pallas as pl
from jax.experimental.pallas import tpu as pltpu
from jax.experimental.pallas import tpu_sc as plsc

EMBED = 32
NC, NS = 2, 16
NW = NC * NS


@functools.lru_cache(maxsize=None)
def _make_gather(batch: int, hist: int):
    bw = batch // NW
    assert bw * NW == batch

    mesh = plsc.VectorSubcoreMesh(
        core_axis_name="c", subcore_axis_name="s",
        num_cores=NC, num_subcores=NS)

    @functools.partial(
        pl.kernel,
        out_type=jax.ShapeDtypeStruct((hist, batch, EMBED), jnp.float32),
        mesh=mesh,
        scratch_types=[
            pltpu.VMEM((2, bw), jnp.int32),
            pltpu.VMEM((2, bw, EMBED), jnp.float32),
            pltpu.SemaphoreType.DMA,
            pltpu.SemaphoreType.DMA,
            pltpu.SemaphoreType.DMA,
            pltpu.SemaphoreType.DMA,
        ],
        compiler_params=pltpu.CompilerParams(use_tc_tiling_on_sc=False),
    )
    def grab(idx_hbm, tab_hbm, out_hbm, idx_v, rows_v, g0, g1, o0, o1):
        wid = lax.axis_index("s") * NC + lax.axis_index("c")
        b0 = wid * bw
        gsem = (g0, g1)
        osem = (o0, o1)

        def stage_and_gather(h, hb):
            pltpu.sync_copy(idx_hbm.at[h, pl.ds(b0, bw)], idx_v.at[hb])
            return pltpu.async_copy(tab_hbm.at[idx_v.at[hb]],
                                    rows_v.at[hb], gsem[hb])

        def fire_out(h, hb):
            return [pltpu.async_copy(rows_v.at[hb],
                                     out_hbm.at[h, pl.ds(b0, bw)],
                                     osem[hb])]

        gd = stage_and_gather(0, 0)
        ods = {}
        for h in range(hist):
            hb = h % 2
            nhb = (hb + 1) % 2
            if h + 1 < hist:
                if h >= 1:
                    for d in ods.pop(h - 1):
                        d.wait()
                ngd = stage_and_gather(h + 1, nhb)
            gd.wait()
            ods[h] = fire_out(h, hb)
            if h + 1 < hist:
                gd = ngd
        for ds_ in ods.values():
            for d in ds_:
                d.wait()

    return grab


def kernel(inputs, embeddings):
    batch, hist = inputs.shape
    idx_t = inputs.T if inputs.dtype == jnp.int32 else inputs.T.astype(jnp.int32)
    out_t = _make_gather(batch, hist)(idx_t, embeddings)
    return out_t.transpose(1, 0, 2)
